# baseline (speedup 1.0000x reference)
import jax
import jax.numpy as jnp
from jax.experimental import pallas as pl

N_NODES = 10000
N_GRAPHS = 64


def _bn(x, gamma, beta, eps=1e-5):
    m = jnp.mean(x, axis=0)
    v = jnp.var(x, axis=0)
    return gamma * (x - m) / jnp.sqrt(v + eps) + beta


def _gcn(x, src, dst, ew, W, b):
    h = x @ W
    deg = jax.ops.segment_sum(ew, dst, num_segments=N_NODES)
    dinv = jnp.where(deg > 0, jax.lax.rsqrt(jnp.maximum(deg, 1e-12)), 0.0)
    norm = dinv[src] * ew * dinv[dst]
    out = jax.ops.segment_sum(norm[:, None] * h[src], dst, num_segments=N_NODES)
    return out + b


def _branch_fwd(p, x, src, dst, ew, batch):
    h = _bn(x, *p['bn0'])
    h = jax.nn.elu(_bn(_gcn(h, src, dst, ew, p['gcn1_W'], p['gcn1_b']), *p['bn1']))
    h = jax.nn.elu(_bn(_gcn(h, src, dst, ew, p['gcn2_W'], p['gcn2_b']), *p['bn2']))
    h = jax.nn.elu(_bn(_gcn(h, src, dst, ew, p['gcn3_W'], p['gcn3_b']), *p['bn3']))
    g = h @ p['gate1_W'] + p['gate1_b']
    g = jax.nn.relu(_bn(g, *p['gate_bn']))
    g = g @ p['gate2_W'] + p['gate2_b']
    gmax = jax.ops.segment_max(g, batch, num_segments=N_GRAPHS)
    gmax = jnp.where(jnp.isfinite(gmax), gmax, 0.0)
    e = jnp.exp(g - gmax[batch])
    denom = jax.ops.segment_sum(e, batch, num_segments=N_GRAPHS)
    attn = e / (denom[batch] + 1e-16)
    pooled = jax.ops.segment_sum(attn * h, batch, num_segments=N_GRAPHS)
    o = jax.nn.elu(_bn(pooled @ p['dnn1_W'] + p['dnn1_b'], *p['dnn_bn']))
    return o @ p['dnn2_W'] + p['dnn2_b']


def _copy_kernel(a_ref, o_ref):
    o_ref[...] = a_ref[...]


def kernel(x, edge_index, edge_weight, batch, params):
    loops = jnp.arange(N_NODES, dtype=edge_index.dtype)
    src = jnp.concatenate([edge_index[0], loops])
    dst = jnp.concatenate([edge_index[1], loops])
    ew = jnp.concatenate([edge_weight, jnp.ones((N_NODES,), dtype=edge_weight.dtype)])
    a_out = _branch_fwd(params['a'], x, src, dst, ew, batch)
    c_out = _branch_fwd(params['c'], x, src, dst, ew, batch)
    a_out = pl.pallas_call(
        _copy_kernel,
        out_shape=jax.ShapeDtypeStruct(a_out.shape, a_out.dtype),
    )(a_out)
    return (a_out, c_out)


# speedup vs baseline: 2.4555x; 2.4555x over previous
import functools

import jax
import jax.numpy as jnp
from jax import lax
from jax.experimental import pallas as pl
from jax.experimental.pallas import tpu as pltpu
from jax.experimental.pallas import tpu_sc as plsc

N_NODES = 10000
N_GRAPHS = 64
N_PAD = 10240
ROWS_PER_TILE = 640
E_PAD = 163840
CHUNK = 128
NCHUNK = 80
NS = 16
NC = 2


def _make_spmm(w):
    mesh = plsc.VectorSubcoreMesh(core_axis_name="c", subcore_axis_name="s")

    @functools.partial(
        pl.kernel,
        mesh=mesh,
        compiler_params=pltpu.CompilerParams(
            needs_layout_passes=False, use_tc_tiling_on_sc=False),
        out_type=[
            jax.ShapeDtypeStruct((N_PAD, w), jnp.float32),
            jax.ShapeDtypeStruct((N_PAD, w), jnp.float32),
        ],
        scratch_types=[
            pltpu.VMEM((NCHUNK, CHUNK), jnp.int32),
            pltpu.VMEM((NCHUNK, CHUNK), jnp.int32),
            pltpu.VMEM((NCHUNK * CHUNK,), jnp.float32),
            pltpu.VMEM((CHUNK, w), jnp.float32),
            pltpu.VMEM_SHARED((N_PAD, w), jnp.float32),
            pltpu.SemaphoreType.DMA,
        ],
    )
    def spmm(za, zc, srcs, dsts, wgts, zeros, pa, pc,
             srcv, dstv, wgtv, gbuf, acc, sem):
        cid = lax.axis_index("c")
        sid = lax.axis_index("s")

        pltpu.sync_copy(srcs.at[sid], srcv)
        pltpu.sync_copy(dsts.at[sid], dstv)
        pltpu.sync_copy(wgts.at[sid], wgtv)
        r0 = sid * ROWS_PER_TILE
        pltpu.sync_copy(zeros.at[pl.ds(r0, ROWS_PER_TILE)],
                        acc.at[pl.ds(r0, ROWS_PER_TILE)])
        plsc.subcore_barrier()

        def run(z_hbm):
            def chunk_body(j, _):
                pltpu.async_copy(z_hbm.at[srcv.at[j]], gbuf, sem).wait()

                def edge_body(e, _):
                    s = plsc.load_gather(
                        wgtv, [jnp.full((16,), j * CHUNK + e, jnp.int32)])
                    for f in range(w // 16):
                        gbuf[e, pl.ds(16 * f, 16)] = (
                            gbuf[e, pl.ds(16 * f, 16)] * s)
                    return 0

                lax.fori_loop(0, CHUNK, edge_body, 0)
                pltpu.sync_copy(gbuf, acc.at[dstv.at[j]], add=True)
                return 0

            lax.fori_loop(0, NCHUNK, chunk_body, 0)

        @pl.when(cid == 0)
        def _():
            run(za)

        @pl.when(cid == 1)
        def _():
            run(zc)

        plsc.subcore_barrier()

        @pl.when(cid == 0)
        def _():
            pltpu.sync_copy(acc.at[pl.ds(r0, ROWS_PER_TILE)],
                            pa.at[pl.ds(r0, ROWS_PER_TILE)])

        @pl.when(cid == 1)
        def _():
            pltpu.sync_copy(acc.at[pl.ds(r0, ROWS_PER_TILE)],
                            pc.at[pl.ds(r0, ROWS_PER_TILE)])

    return spmm


_spmm32 = _make_spmm(32)
_spmm64 = _make_spmm(64)
_spmm128 = _make_spmm(128)
_SPMM = {32: _spmm32, 64: _spmm64, 128: _spmm128}


def _pad_edges(a, mode='2d'):
    a = jnp.concatenate([a, jnp.zeros((E_PAD - a.shape[0],), a.dtype)])
    if mode == 'flat':
        return a.reshape(NS, NCHUNK * CHUNK)
    return a.reshape(NS, NCHUNK, CHUNK)


def _bn(x, gamma, beta, eps=1e-5):
    m = jnp.mean(x, axis=0)
    v = jnp.var(x, axis=0)
    return gamma * (x - m) / jnp.sqrt(v + eps) + beta


def _branch_tail(p, h, batch):
    g = h @ p['gate1_W'] + p['gate1_b']
    g = jax.nn.relu(_bn(g, *p['gate_bn']))
    g = g @ p['gate2_W'] + p['gate2_b']
    gmax = jax.ops.segment_max(g, batch, num_segments=N_GRAPHS)
    gmax = jnp.where(jnp.isfinite(gmax), gmax, 0.0)
    e = jnp.exp(g - gmax[batch])
    denom = jax.ops.segment_sum(e, batch, num_segments=N_GRAPHS)
    attn = e / (denom[batch] + 1e-16)
    pooled = jax.ops.segment_sum(attn * h, batch, num_segments=N_GRAPHS)
    o = jax.nn.elu(_bn(pooled @ p['dnn1_W'] + p['dnn1_b'], *p['dnn_bn']))
    return o @ p['dnn2_W'] + p['dnn2_b']


def kernel(x, edge_index, edge_weight, batch, params):
    src = edge_index[0]
    dst = edge_index[1]
    ew = edge_weight

    deg = jax.ops.segment_sum(ew, dst, num_segments=N_NODES) + 1.0
    dinv = jax.lax.rsqrt(deg)
    norm = dinv[src] * ew * dinv[dst]
    invdeg = (dinv * dinv)[:, None]

    srcs = _pad_edges(src)
    dsts = _pad_edges(dst)
    norms = _pad_edges(norm, mode='flat')

    pa, pc = params['a'], params['c']
    ha = _bn(x, *pa['bn0'])
    hc = _bn(x, *pc['bn0'])

    for li, (wn, wi) in enumerate([('gcn1', 32), ('gcn2', 64), ('gcn3', 128)]):
        za = ha @ pa[wn + '_W']
        zc = hc @ pc[wn + '_W']
        zap = jnp.concatenate([za, jnp.zeros((N_PAD - N_NODES, wi), jnp.float32)])
        zcp = jnp.concatenate([zc, jnp.zeros((N_PAD - N_NODES, wi), jnp.float32)])
        zeros = jnp.zeros((N_PAD, wi), jnp.float32)
        agg_a, agg_c = _SPMM[wi](zap, zcp, srcs, dsts, norms, zeros)
        ya = agg_a[:N_NODES] + invdeg * za + pa[wn + '_b']
        yc = agg_c[:N_NODES] + invdeg * zc + pc[wn + '_b']
        bn_name = 'bn%d' % (li + 1)
        ha = jax.nn.elu(_bn(ya, *pa[bn_name]))
        hc = jax.nn.elu(_bn(yc, *pc[bn_name]))

    a_out = _branch_tail(pa, ha, batch)
    c_out = _branch_tail(pc, hc, batch)
    return (a_out, c_out)


# speedup vs baseline: 6.2565x; 2.5480x over previous
import functools

import jax
import jax.numpy as jnp
from jax import lax
from jax.experimental import pallas as pl
from jax.experimental.pallas import tpu as pltpu
from jax.experimental.pallas import tpu_sc as plsc

N_NODES = 10000
N_GRAPHS = 64
N_PAD = 10240
RPT = 640
E_PAD = 163840
CHUNK = 128
NCHUNK = 80
EPT = NCHUNK * CHUNK
NS = 16
NC = 2
EPS = 1e-5

_SC_PARAMS = pltpu.CompilerParams(
    needs_layout_passes=False, use_tc_tiling_on_sc=False)


def _full(v, dtype=jnp.float32):
    return jnp.full((16,), v, dtype)


def _newton_rsqrt(x):
    i = plsc.bitcast(x, jnp.int32)
    i = _full(0x5F3759DF, jnp.int32) - lax.shift_right_logical(i, 1)
    y = plsc.bitcast(i, jnp.float32)
    for _ in range(3):
        y = y * (_full(1.5) - _full(0.5) * x * y * y)
    return y


def _scale_chunk(gbuf, wgt_flat, j, w):
    def edge_body(e, _):
        s = plsc.load_gather(
            wgt_flat, [jnp.full((16,), j * CHUNK + e, jnp.int32)])
        for f in range(w // 16):
            gbuf[e, pl.ds(16 * f, 16)] = gbuf[e, pl.ds(16 * f, 16)] * s
        return 0
    lax.fori_loop(0, CHUNK, edge_body, 0)


def _make_spmm(w):
    mesh = plsc.VectorSubcoreMesh(core_axis_name="c", subcore_axis_name="s")

    @functools.partial(
        pl.kernel,
        mesh=mesh,
        compiler_params=_SC_PARAMS,
        out_type=[
            jax.ShapeDtypeStruct((N_PAD, w), jnp.float32),
            jax.ShapeDtypeStruct((N_PAD, w), jnp.float32),
        ],
        scratch_types=[
            pltpu.VMEM((NCHUNK, CHUNK), jnp.int32),
            pltpu.VMEM((NCHUNK, CHUNK), jnp.int32),
            pltpu.VMEM((EPT,), jnp.float32),
            pltpu.VMEM((CHUNK, w), jnp.float32),
            pltpu.VMEM_SHARED((N_PAD, w), jnp.float32),
            pltpu.SemaphoreType.DMA,
        ],
    )
    def spmm(za, zc, srcs, dsts, wgts, zeros, pa, pc,
             srcv, dstv, wgtv, gbuf, acc, sem):
        cid = lax.axis_index("c")
        sid = lax.axis_index("s")

        pltpu.sync_copy(srcs.at[sid], srcv)
        pltpu.sync_copy(dsts.at[sid], dstv)
        pltpu.sync_copy(wgts.at[sid], wgtv)
        r0 = sid * RPT
        pltpu.sync_copy(zeros.at[pl.ds(r0, RPT)], acc.at[pl.ds(r0, RPT)])
        plsc.subcore_barrier()

        def run(z_hbm):
            def chunk_body(j, _):
                pltpu.async_copy(z_hbm.at[srcv.at[j]], gbuf, sem).wait()
                _scale_chunk(gbuf, wgtv, j, w)
                pltpu.sync_copy(gbuf, acc.at[dstv.at[j]], add=True)
                return 0
            lax.fori_loop(0, NCHUNK, chunk_body, 0)

        @pl.when(cid == 0)
        def _():
            run(za)

        @pl.when(cid == 1)
        def _():
            run(zc)

        plsc.subcore_barrier()

        @pl.when(cid == 0)
        def _():
            pltpu.sync_copy(acc.at[pl.ds(r0, RPT)], pa.at[pl.ds(r0, RPT)])

        @pl.when(cid == 1)
        def _():
            pltpu.sync_copy(acc.at[pl.ds(r0, RPT)], pc.at[pl.ds(r0, RPT)])

    return spmm


def _make_spmm1():
    w = 32
    mesh = plsc.VectorSubcoreMesh(core_axis_name="c", subcore_axis_name="s")

    @functools.partial(
        pl.kernel,
        mesh=mesh,
        compiler_params=_SC_PARAMS,
        out_type=[
            jax.ShapeDtypeStruct((N_PAD, w), jnp.float32),
            jax.ShapeDtypeStruct((N_PAD, w), jnp.float32),
            jax.ShapeDtypeStruct((N_PAD, 16), jnp.float32),
            jax.ShapeDtypeStruct((NS, EPT), jnp.float32),
        ],
        scratch_types=[
            pltpu.VMEM((NCHUNK, CHUNK), jnp.int32),
            pltpu.VMEM((NCHUNK, CHUNK), jnp.int32),
            pltpu.VMEM((EPT,), jnp.float32),
            pltpu.VMEM((EPT,), jnp.float32),
            pltpu.VMEM((CHUNK, w), jnp.float32),
            pltpu.VMEM((CHUNK, 16), jnp.float32),
            pltpu.VMEM((RPT, 16), jnp.float32),
            pltpu.VMEM((RPT,), jnp.float32),
            pltpu.VMEM((N_PAD,), jnp.float32),
            pltpu.VMEM_SHARED((N_PAD, w), jnp.float32),
            pltpu.VMEM_SHARED((N_PAD, 16), jnp.float32),
            pltpu.VMEM_SHARED((N_PAD,), jnp.float32),
            pltpu.SemaphoreType.DMA,
        ],
    )
    def spmm1(za, zc, srcs, dsts, ews, zeros32, zeros16, pa, pc, degout,
              normout, srcv, dstv, ewv, normv, gbuf, degbuf, degv, dinvp,
              dinvv, acc, degacc, dinvsp, sem):
        cid = lax.axis_index("c")
        sid = lax.axis_index("s")

        pltpu.sync_copy(srcs.at[sid], srcv)
        pltpu.sync_copy(dsts.at[sid], dstv)
        pltpu.sync_copy(ews.at[sid], ewv)
        r0 = sid * RPT
        pltpu.sync_copy(zeros32.at[pl.ds(r0, RPT)], acc.at[pl.ds(r0, RPT)])
        pltpu.sync_copy(zeros16.at[pl.ds(r0, RPT)], degacc.at[pl.ds(r0, RPT)])
        plsc.subcore_barrier()

        def deg_chunk(j, _):
            def deg_edge(e, _):
                s = plsc.load_gather(
                    ewv, [jnp.full((16,), j * CHUNK + e, jnp.int32)])
                degbuf[e, :] = s
                return 0
            lax.fori_loop(0, CHUNK, deg_edge, 0)
            pltpu.sync_copy(degbuf, degacc.at[dstv.at[j]], add=True)
            return 0
        lax.fori_loop(0, NCHUNK, deg_chunk, 0)
        plsc.subcore_barrier()

        pltpu.sync_copy(degacc.at[pl.ds(r0, RPT)], degv)
        lane0 = lax.iota(jnp.int32, 16) == 0

        def dinv_row(r, _):
            d = degv[r, :] + _full(1.0)
            y = _newton_rsqrt(d)
            plsc.store_scatter(dinvp, [jnp.full((16,), r, jnp.int32)], y,
                               mask=lane0)
            return 0
        lax.fori_loop(0, RPT, dinv_row, 0)
        pltpu.sync_copy(dinvp, dinvsp.at[pl.ds(r0, RPT)])
        plsc.subcore_barrier()
        pltpu.sync_copy(dinvsp, dinvv)

        def norm_chunk(j, _):
            for g in range(CHUNK // 16):
                s16 = srcv[j, pl.ds(16 * g, 16)]
                d16 = dstv[j, pl.ds(16 * g, 16)]
                e16 = ewv[pl.ds(j * CHUNK + 16 * g, 16)]
                a16 = plsc.load_gather(dinvv, [s16])
                b16 = plsc.load_gather(dinvv, [d16])
                normv[pl.ds(j * CHUNK + 16 * g, 16)] = a16 * e16 * b16
            return 0
        lax.fori_loop(0, NCHUNK, norm_chunk, 0)

        def run(z_hbm):
            def chunk_body(j, _):
                pltpu.async_copy(z_hbm.at[srcv.at[j]], gbuf, sem).wait()
                _scale_chunk(gbuf, normv, j, w)
                pltpu.sync_copy(gbuf, acc.at[dstv.at[j]], add=True)
                return 0
            lax.fori_loop(0, NCHUNK, chunk_body, 0)

        @pl.when(cid == 0)
        def _():
            run(za)

        @pl.when(cid == 1)
        def _():
            run(zc)

        plsc.subcore_barrier()

        @pl.when(cid == 0)
        def _():
            pltpu.sync_copy(acc.at[pl.ds(r0, RPT)], pa.at[pl.ds(r0, RPT)])
            pltpu.sync_copy(degacc.at[pl.ds(r0, RPT)],
                            degout.at[pl.ds(r0, RPT)])
            pltpu.sync_copy(normv, normout.at[sid])

        @pl.when(cid == 1)
        def _():
            pltpu.sync_copy(acc.at[pl.ds(r0, RPT)], pc.at[pl.ds(r0, RPT)])

    return spmm1


_SPMM1 = _make_spmm1()
_SPMM = {64: _make_spmm(64), 128: _make_spmm(128)}


def _bn_stats(y):
    m = jnp.mean(y, axis=0, keepdims=True)
    v = jnp.mean(jnp.square(y - m), axis=0, keepdims=True)
    return m, v


def _bn_apply(y, m, v, gamma, beta):
    return gamma * (y - m) * lax.rsqrt(v + EPS) + beta


def _elu(y):
    return jnp.where(y > 0, y, jnp.exp(jnp.minimum(y, 0.0)) - 1.0)


def _pad_rows(z, w):
    return jnp.concatenate([z, jnp.zeros((N_PAD - N_NODES, w), jnp.float32)])


def _t0_body(x_ref, ga_ref, ba_ref, wa_ref, gc_ref, bc_ref, wc_ref,
             za_ref, zc_ref):
    x = x_ref[...]
    m, v = _bn_stats(x)
    rs = lax.rsqrt(v + EPS)
    xn = (x - m) * rs
    za = (ga_ref[...] * xn + ba_ref[...]) @ wa_ref[...]
    zc = (gc_ref[...] * xn + bc_ref[...]) @ wc_ref[...]
    za_ref[...] = _pad_rows(za, za.shape[1])
    zc_ref[...] = _pad_rows(zc, zc.shape[1])


def _tmid_body(pa_ref, pc_ref, za_ref, zc_ref, deg_ref,
               b_a, g_a, be_a, w_a, b_c, g_c, be_c, w_c,
             oa_ref, oc_ref):
    invdeg = 1.0 / (deg_ref[:N_NODES, 0:1] + 1.0)

    def one(p_ref, z_ref, bias, gamma, beta, wnext):
        z = z_ref[:N_NODES]
        y = p_ref[:N_NODES] + invdeg * z + bias[...]
        m, v = _bn_stats(y)
        h = _elu(_bn_apply(y, m, v, gamma[...], beta[...]))
        return h @ wnext[...]

    oa = one(pa_ref, za_ref, b_a, g_a, be_a, w_a)
    oc = one(pc_ref, zc_ref, b_c, g_c, be_c, w_c)
    oa_ref[...] = _pad_rows(oa, oa.shape[1])
    oc_ref[...] = _pad_rows(oc, oc.shape[1])


_TB = 1000
_NB = N_NODES // _TB
_FN = float(N_NODES)


def _t3_body(p_ref, z_ref, deg_ref, batch_ref,
             b3, g3, be3, w_g1, b_g1, g_gbn, be_gbn, w_g2, b_g2,
             w_d1, b_d1, g_dbn, be_dbn, w_d2, b_d2,
             out_ref,
             h_scr, g_scr, ysum, yvar, g1sum, g1var, gmax_s, den_s, num_s):
    ph = pl.program_id(0)
    b = pl.program_id(1)
    neg = jnp.float32(-1e30)

    def y_block():
        invdeg = 1.0 / (deg_ref[:, 0:1] + 1.0)
        return p_ref[...] + invdeg * z_ref[...] + b3[...]

    def mask():
        iota_g = lax.broadcasted_iota(jnp.int32, (N_GRAPHS, _TB), 0)
        return batch_ref[...].reshape(1, _TB) == iota_g

    def gate_pre(h):
        return h @ w_g1[...] + b_g1[...]

    @pl.when((ph == 0) & (b == 0))
    def _():
        ysum[...] = jnp.zeros_like(ysum)
        yvar[...] = jnp.zeros_like(yvar)
        g1sum[...] = jnp.zeros_like(g1sum)
        g1var[...] = jnp.zeros_like(g1var)
        gmax_s[...] = jnp.full_like(gmax_s, neg)
        den_s[...] = jnp.zeros_like(den_s)
        num_s[...] = jnp.zeros_like(num_s)

    @pl.when(ph == 0)
    def _():
        ysum[...] += jnp.sum(y_block(), axis=0, keepdims=True)

    @pl.when(ph == 1)
    def _():
        m = ysum[...] / _FN
        yvar[...] += jnp.sum(jnp.square(y_block() - m), axis=0, keepdims=True)

    @pl.when(ph == 2)
    def _():
        m = ysum[...] / _FN
        v = yvar[...] / _FN
        h = _elu(_bn_apply(y_block(), m, v, g3[...], be3[...]))
        h_scr[pl.ds(b * _TB, _TB), :] = h
        g1sum[...] += jnp.sum(gate_pre(h), axis=0, keepdims=True)

    @pl.when(ph == 3)
    def _():
        g1 = gate_pre(h_scr[pl.ds(b * _TB, _TB), :])
        m1 = g1sum[...] / _FN
        g1var[...] += jnp.sum(jnp.square(g1 - m1), axis=0, keepdims=True)

    @pl.when(ph == 4)
    def _():
        g1 = gate_pre(h_scr[pl.ds(b * _TB, _TB), :])
        m1 = g1sum[...] / _FN
        v1 = g1var[...] / _FN
        g1 = jnp.maximum(_bn_apply(g1, m1, v1, g_gbn[...], be_gbn[...]), 0.0)
        g = g1 @ w_g2[...] + b_g2[...]
        g_scr[pl.ds(b * _TB, _TB), :] = g
        gt = g.reshape(1, _TB)
        cand = jnp.max(jnp.where(mask(), gt, neg), axis=1, keepdims=True)
        gmax_s[...] = jnp.maximum(gmax_s[...], cand)

    @pl.when(ph == 5)
    def _():
        gt = g_scr[pl.ds(b * _TB, _TB), :].reshape(1, _TB)
        e = jnp.exp(jnp.where(mask(), gt - gmax_s[...], neg))
        den_s[...] += jnp.sum(e, axis=1, keepdims=True)
        num_s[...] += jnp.dot(e, h_scr[pl.ds(b * _TB, _TB), :],
                              preferred_element_type=jnp.float32)

    @pl.when((ph == 6) & (b == 0))
    def _():
        pooled = num_s[...] / (den_s[...] + 1e-16)
        o = pooled @ w_d1[...] + b_d1[...]
        mo, vo = _bn_stats(o)
        o = _elu(_bn_apply(o, mo, vo, g_dbn[...], be_dbn[...]))
        out_ref[...] = o @ w_d2[...] + b_d2[...]


def _t3_call(p3, z3, deg, batch2d, prm):
    blk = lambda w: pl.BlockSpec((_TB, w), lambda ph, b: (b, 0))
    full = lambda s: pl.BlockSpec(s, lambda ph, b: (0, 0))
    param_shapes = [a.shape for a in prm]
    return pl.pallas_call(
        _t3_body,
        grid=(7, _NB),
        in_specs=[blk(128), blk(128), blk(16),
                  pl.BlockSpec((1, 1, _TB), lambda ph, b: (b, 0, 0))]
                 + [full(s) for s in param_shapes],
        out_specs=full((N_GRAPHS, 32)),
        out_shape=jax.ShapeDtypeStruct((N_GRAPHS, 32), jnp.float32),
        scratch_shapes=[
            pltpu.VMEM((N_NODES, 128), jnp.float32),
            pltpu.VMEM((N_NODES, 1), jnp.float32),
            pltpu.VMEM((1, 128), jnp.float32),
            pltpu.VMEM((1, 128), jnp.float32),
            pltpu.VMEM((1, 42), jnp.float32),
            pltpu.VMEM((1, 42), jnp.float32),
            pltpu.VMEM((N_GRAPHS, 1), jnp.float32),
            pltpu.VMEM((N_GRAPHS, 1), jnp.float32),
            pltpu.VMEM((N_GRAPHS, 128), jnp.float32),
        ],
    )(p3, z3, deg, batch2d, *prm)


def _row(a):
    return a.reshape(1, -1)


def _pad_edges(a, mode='2d'):
    a = jnp.concatenate([a, jnp.zeros((E_PAD - a.shape[0],), a.dtype)])
    if mode == 'flat':
        return a.reshape(NS, EPT)
    return a.reshape(NS, NCHUNK, CHUNK)


def kernel(x, edge_index, edge_weight, batch, params):
    srcs = _pad_edges(edge_index[0])
    dsts = _pad_edges(edge_index[1])
    ews = _pad_edges(edge_weight, mode='flat')

    pa, pc = params['a'], params['c']

    z1a, z1c = pl.pallas_call(
        _t0_body,
        out_shape=[jax.ShapeDtypeStruct((N_PAD, 32), jnp.float32)] * 2,
    )(x, _row(pa['bn0'][0]), _row(pa['bn0'][1]), pa['gcn1_W'],
      _row(pc['bn0'][0]), _row(pc['bn0'][1]), pc['gcn1_W'])

    zeros32 = jnp.zeros((N_PAD, 32), jnp.float32)
    zeros16 = jnp.zeros((N_PAD, 16), jnp.float32)
    p1a, p1c, deg, norms = _SPMM1(z1a, z1c, srcs, dsts, ews,
                                  zeros32, zeros16)

    def tmid(p1, p2, w, pA, pB, bnname, wname):
        return pl.pallas_call(
            _tmid_body,
            out_shape=[jax.ShapeDtypeStruct((N_PAD, w), jnp.float32)] * 2,
        )(p1[0], p1[1], p2[0], p2[1], deg,
          _row(pA[bnname.replace('bn', 'gcn') + '_b']),
          _row(pA[bnname][0]), _row(pA[bnname][1]), pA[wname],
          _row(pB[bnname.replace('bn', 'gcn') + '_b']),
          _row(pB[bnname][0]), _row(pB[bnname][1]), pB[wname])

    z2a, z2c = tmid((p1a, p1c), (z1a, z1c), 64, pa, pc, 'bn1', 'gcn2_W')
    zeros64 = jnp.zeros((N_PAD, 64), jnp.float32)
    p2a, p2c = _SPMM[64](z2a, z2c, srcs, dsts, norms, zeros64)

    z3a, z3c = tmid((p2a, p2c), (z2a, z2c), 128, pa, pc, 'bn2', 'gcn3_W')
    zeros128 = jnp.zeros((N_PAD, 128), jnp.float32)
    p3a, p3c = _SPMM[128](z3a, z3c, srcs, dsts, norms, zeros128)

    def t3_params(p):
        return (
            _row(p['gcn3_b']), _row(p['bn3'][0]), _row(p['bn3'][1]),
            p['gate1_W'], _row(p['gate1_b']),
            _row(p['gate_bn'][0]), _row(p['gate_bn'][1]),
            p['gate2_W'], _row(p['gate2_b']),
            p['dnn1_W'], _row(p['dnn1_b']),
            _row(p['dnn_bn'][0]), _row(p['dnn_bn'][1]),
            p['dnn2_W'], _row(p['dnn2_b']),
        )

    batch2d = batch.reshape(_NB, 1, _TB)
    a_out = _t3_call(p3a, z3a, deg, batch2d, t3_params(pa))
    c_out = _t3_call(p3c, z3c, deg, batch2d, t3_params(pc))

    return (a_out, c_out)


# speedup vs baseline: 8.0146x; 1.2810x over previous
import functools

import jax
import jax.numpy as jnp
from jax import lax
from jax.experimental import pallas as pl
from jax.experimental.pallas import tpu as pltpu
from jax.experimental.pallas import tpu_sc as plsc

N_NODES = 10000
N_GRAPHS = 64
N_PAD = 10240
RPT = 640
E_PAD = 163840
CHUNK = 128
NCHUNK = 80
EPT = NCHUNK * CHUNK
NS = 16
NC = 2
EPS = 1e-5

_SC_PARAMS = pltpu.CompilerParams(
    needs_layout_passes=False, use_tc_tiling_on_sc=False)


def _full(v, dtype=jnp.float32):
    return jnp.full((16,), v, dtype)


def _newton_rsqrt(x):
    i = plsc.bitcast(x, jnp.int32)
    i = _full(0x5F3759DF, jnp.int32) - lax.shift_right_logical(i, 1)
    y = plsc.bitcast(i, jnp.float32)
    for _ in range(3):
        y = y * (_full(1.5) - _full(0.5) * x * y * y)
    return y


def _scale_chunk(gbuf, wgt_flat, j, w):
    @plsc.parallel_loop(0, CHUNK, unroll=8)
    def _(e):
        s = plsc.load_gather(
            wgt_flat, [jnp.full((16,), j * CHUNK + e, jnp.int32)])
        for f in range(w // 16):
            gbuf[e, pl.ds(16 * f, 16)] = gbuf[e, pl.ds(16 * f, 16)] * s


_NBUF = 4


def _pipelined_agg(z_hbm, srcv, dstv, wgtv, acc, bufs, gsems, ssems, w):
    npair = NCHUNK // _NBUF
    pltpu.async_copy(z_hbm.at[srcv.at[0]], bufs[0], gsems[0])

    def body(j2, _):
        for b in range(_NBUF):
            j = j2 * _NBUF + b
            pltpu.make_async_copy(z_hbm.at[srcv.at[j]], bufs[b],
                                  gsems[b]).wait()
            bn = (b + 1) % _NBUF
            if b == _NBUF - 1:
                @pl.when(j2 < npair - 1)
                def _():
                    pltpu.make_async_copy(
                        bufs[bn], acc.at[dstv.at[j - (_NBUF - 1)]],
                        ssems[bn]).wait()
                    pltpu.async_copy(z_hbm.at[srcv.at[j + 1]], bufs[bn],
                                     gsems[bn])
            else:
                @pl.when(j2 > 0)
                def _():
                    pltpu.make_async_copy(
                        bufs[bn], acc.at[dstv.at[j - (_NBUF - 1)]],
                        ssems[bn]).wait()
                pltpu.async_copy(z_hbm.at[srcv.at[j + 1]], bufs[bn],
                                 gsems[bn])
            _scale_chunk(bufs[b], wgtv, j, w)
            pltpu.async_copy(bufs[b], acc.at[dstv.at[j]], ssems[b],
                             add=True)
        return 0

    lax.fori_loop(0, npair, body, 0)
    for b in range(_NBUF):
        j = (npair - 1) * _NBUF + b
        pltpu.make_async_copy(bufs[b], acc.at[dstv.at[j]], ssems[b]).wait()


def _make_spmm(w, nbuf):
    mesh = plsc.VectorSubcoreMesh(core_axis_name="c", subcore_axis_name="s")

    @functools.partial(
        pl.kernel,
        mesh=mesh,
        compiler_params=_SC_PARAMS,
        out_type=[
            jax.ShapeDtypeStruct((N_PAD, w), jnp.float32),
            jax.ShapeDtypeStruct((N_PAD, w), jnp.float32),
        ],
        scratch_types=[
            pltpu.VMEM((NCHUNK, CHUNK), jnp.int32),
            pltpu.VMEM_SHARED((N_PAD, w), jnp.float32),
        ] + [pltpu.VMEM((CHUNK, w), jnp.float32)] * nbuf
          + [pltpu.VMEM((CHUNK,), jnp.int32)] * nbuf
          + [pltpu.VMEM((CHUNK,), jnp.float32)] * nbuf
          + [pltpu.SemaphoreType.DMA] * (2 * nbuf),
    )
    def spmm(za, zc, srcs, dsts, wgts, zeros, pa, pc,
             srcv, acc, *bufsem):
        bufs = bufsem[:nbuf]
        dstcs = bufsem[nbuf:2 * nbuf]
        wgtcs = bufsem[2 * nbuf:3 * nbuf]
        gsems = bufsem[3 * nbuf:4 * nbuf]
        ssems = bufsem[4 * nbuf:]
        cid = lax.axis_index("c")
        sid = lax.axis_index("s")

        pltpu.sync_copy(srcs.at[sid], srcv)
        r0 = sid * RPT
        pltpu.sync_copy(zeros.at[pl.ds(r0, RPT)], acc.at[pl.ds(r0, RPT)])
        plsc.subcore_barrier()

        def run(z_hbm):
            npair = NCHUNK // nbuf

            def load(j, b):
                pltpu.async_copy(z_hbm.at[srcv.at[j]], bufs[b], gsems[b])
                pltpu.async_copy(dsts.at[sid].at[j], dstcs[b], gsems[b])
                pltpu.async_copy(wgts.at[sid].at[pl.ds(j * CHUNK, CHUNK)],
                                 wgtcs[b], gsems[b])

            def wait_load(j, b):
                pltpu.make_async_copy(z_hbm.at[srcv.at[j]], bufs[b],
                                      gsems[b]).wait()
                pltpu.make_async_copy(dsts.at[sid].at[j], dstcs[b],
                                      gsems[b]).wait()
                pltpu.make_async_copy(
                    wgts.at[sid].at[pl.ds(j * CHUNK, CHUNK)], wgtcs[b],
                    gsems[b]).wait()

            def wait_scatter(b):
                pltpu.make_async_copy(bufs[b], acc.at[dstcs[b]],
                                      ssems[b]).wait()

            load(0, 0)

            def body(j2, _):
                for b in range(nbuf):
                    j = j2 * nbuf + b
                    wait_load(j, b)
                    bn = (b + 1) % nbuf
                    if b == nbuf - 1:
                        @pl.when(j2 < npair - 1)
                        def _():
                            wait_scatter(bn)
                            load(j + 1, bn)
                    else:
                        @pl.when(j2 > 0)
                        def _():
                            wait_scatter(bn)
                        load(j + 1, bn)
                    _scale_chunk(bufs[b], wgtcs[b], 0, w)
                    pltpu.async_copy(bufs[b], acc.at[dstcs[b]], ssems[b],
                                     add=True)
                return 0

            lax.fori_loop(0, npair, body, 0)
            for b in range(nbuf):
                wait_scatter(b)

        @pl.when(cid == 0)
        def _():
            run(za)

        @pl.when(cid == 1)
        def _():
            run(zc)

        plsc.subcore_barrier()

        @pl.when(cid == 0)
        def _():
            pltpu.sync_copy(acc.at[pl.ds(r0, RPT)], pa.at[pl.ds(r0, RPT)])

        @pl.when(cid == 1)
        def _():
            pltpu.sync_copy(acc.at[pl.ds(r0, RPT)], pc.at[pl.ds(r0, RPT)])

    return spmm


def _make_spmm1():
    w = 32
    mesh = plsc.VectorSubcoreMesh(core_axis_name="c", subcore_axis_name="s")

    @functools.partial(
        pl.kernel,
        mesh=mesh,
        compiler_params=_SC_PARAMS,
        out_type=[
            jax.ShapeDtypeStruct((N_PAD, w), jnp.float32),
            jax.ShapeDtypeStruct((N_PAD, w), jnp.float32),
            jax.ShapeDtypeStruct((N_PAD, 16), jnp.float32),
            jax.ShapeDtypeStruct((NS, EPT), jnp.float32),
        ],
        scratch_types=[
            pltpu.VMEM((NCHUNK, CHUNK), jnp.int32),
            pltpu.VMEM((NCHUNK, CHUNK), jnp.int32),
            pltpu.VMEM((EPT,), jnp.float32),
            pltpu.VMEM((EPT,), jnp.float32),
            pltpu.VMEM((CHUNK, 16), jnp.float32),
            pltpu.VMEM((RPT, 16), jnp.float32),
            pltpu.VMEM((RPT,), jnp.float32),
            pltpu.VMEM((N_PAD,), jnp.float32),
            pltpu.VMEM_SHARED((N_PAD, w), jnp.float32),
            pltpu.VMEM_SHARED((N_PAD, 16), jnp.float32),
            pltpu.VMEM_SHARED((N_PAD,), jnp.float32),
        ] + [pltpu.VMEM((CHUNK, w), jnp.float32)] * _NBUF
          + [pltpu.SemaphoreType.DMA] * (2 * _NBUF + 1),
    )
    def spmm1(za, zc, srcs, dsts, ews, zeros32, zeros16, pa, pc, degout,
              normout, srcv, dstv, ewv, normv, degbuf, degv, dinvp,
              dinvv, acc, degacc, dinvsp, *bufsem):
        bufs = bufsem[:_NBUF]
        gsems = bufsem[_NBUF:2 * _NBUF]
        ssems = bufsem[2 * _NBUF:3 * _NBUF]
        sem = bufsem[3 * _NBUF]
        cid = lax.axis_index("c")
        sid = lax.axis_index("s")

        pltpu.sync_copy(srcs.at[sid], srcv)
        pltpu.sync_copy(dsts.at[sid], dstv)
        pltpu.sync_copy(ews.at[sid], ewv)
        r0 = sid * RPT
        pltpu.sync_copy(zeros32.at[pl.ds(r0, RPT)], acc.at[pl.ds(r0, RPT)])
        pltpu.sync_copy(zeros16.at[pl.ds(r0, RPT)], degacc.at[pl.ds(r0, RPT)])
        plsc.subcore_barrier()

        def deg_chunk(j, _):
            def deg_edge(e, _):
                s = plsc.load_gather(
                    ewv, [jnp.full((16,), j * CHUNK + e, jnp.int32)])
                degbuf[e, :] = s
                return 0
            lax.fori_loop(0, CHUNK, deg_edge, 0)
            pltpu.sync_copy(degbuf, degacc.at[dstv.at[j]], add=True)
            return 0
        lax.fori_loop(0, NCHUNK, deg_chunk, 0)
        plsc.subcore_barrier()

        pltpu.sync_copy(degacc.at[pl.ds(r0, RPT)], degv)
        lane0 = lax.iota(jnp.int32, 16) == 0

        def dinv_row(r, _):
            d = degv[r, :] + _full(1.0)
            y = _newton_rsqrt(d)
            plsc.store_scatter(dinvp, [jnp.full((16,), r, jnp.int32)], y,
                               mask=lane0)
            return 0
        lax.fori_loop(0, RPT, dinv_row, 0)
        pltpu.sync_copy(dinvp, dinvsp.at[pl.ds(r0, RPT)])
        plsc.subcore_barrier()
        pltpu.sync_copy(dinvsp, dinvv)

        def norm_chunk(j, _):
            for g in range(CHUNK // 16):
                s16 = srcv[j, pl.ds(16 * g, 16)]
                d16 = dstv[j, pl.ds(16 * g, 16)]
                e16 = ewv[pl.ds(j * CHUNK + 16 * g, 16)]
                a16 = plsc.load_gather(dinvv, [s16])
                b16 = plsc.load_gather(dinvv, [d16])
                normv[pl.ds(j * CHUNK + 16 * g, 16)] = a16 * e16 * b16
            return 0
        lax.fori_loop(0, NCHUNK, norm_chunk, 0)

        def run(z_hbm):
            _pipelined_agg(z_hbm, srcv, dstv, normv, acc, bufs, gsems,
                           ssems, w)

        @pl.when(cid == 0)
        def _():
            run(za)

        @pl.when(cid == 1)
        def _():
            run(zc)

        plsc.subcore_barrier()

        @pl.when(cid == 0)
        def _():
            pltpu.sync_copy(acc.at[pl.ds(r0, RPT)], pa.at[pl.ds(r0, RPT)])
            pltpu.sync_copy(degacc.at[pl.ds(r0, RPT)],
                            degout.at[pl.ds(r0, RPT)])
            pltpu.sync_copy(normv, normout.at[sid])

        @pl.when(cid == 1)
        def _():
            pltpu.sync_copy(acc.at[pl.ds(r0, RPT)], pc.at[pl.ds(r0, RPT)])

    return spmm1


_SPMM1 = _make_spmm1()
_SPMM = {64: _make_spmm(64, 4), 128: _make_spmm(128, 2)}


def _bn_stats(y):
    m = jnp.mean(y, axis=0, keepdims=True)
    v = jnp.mean(jnp.square(y - m), axis=0, keepdims=True)
    return m, v


def _bn_apply(y, m, v, gamma, beta):
    return gamma * (y - m) * lax.rsqrt(v + EPS) + beta


def _elu(y):
    return jnp.where(y > 0, y, jnp.exp(jnp.minimum(y, 0.0)) - 1.0)


def _pad_rows(z, w):
    return jnp.concatenate([z, jnp.zeros((N_PAD - N_NODES, w), jnp.float32)])


def _t0_body(x_ref, ga_ref, ba_ref, wa_ref, gc_ref, bc_ref, wc_ref,
             za_ref, zc_ref):
    x = x_ref[...]
    m, v = _bn_stats(x)
    rs = lax.rsqrt(v + EPS)
    xn = (x - m) * rs
    za = (ga_ref[...] * xn + ba_ref[...]) @ wa_ref[...]
    zc = (gc_ref[...] * xn + bc_ref[...]) @ wc_ref[...]
    za_ref[...] = _pad_rows(za, za.shape[1])
    zc_ref[...] = _pad_rows(zc, zc.shape[1])


def _tmid_body(pa_ref, pc_ref, za_ref, zc_ref, deg_ref,
               b_a, g_a, be_a, w_a, b_c, g_c, be_c, w_c,
             oa_ref, oc_ref):
    invdeg = 1.0 / (deg_ref[:N_NODES, 0:1] + 1.0)

    def one(p_ref, z_ref, bias, gamma, beta, wnext):
        z = z_ref[:N_NODES]
        y = p_ref[:N_NODES] + invdeg * z + bias[...]
        m, v = _bn_stats(y)
        h = _elu(_bn_apply(y, m, v, gamma[...], beta[...]))
        return h @ wnext[...]

    oa = one(pa_ref, za_ref, b_a, g_a, be_a, w_a)
    oc = one(pc_ref, zc_ref, b_c, g_c, be_c, w_c)
    oa_ref[...] = _pad_rows(oa, oa.shape[1])
    oc_ref[...] = _pad_rows(oc, oc.shape[1])


_TB = 1000
_NB = N_NODES // _TB
_FN = float(N_NODES)


def _t3_body(p_ref, z_ref, deg_ref, batch_ref,
             b3, g3, be3, w_g1, b_g1, g_gbn, be_gbn, w_g2, b_g2,
             w_d1, b_d1, g_dbn, be_dbn, w_d2, b_d2,
             out_ref,
             h_scr, g_scr, ysum, yvar, g1sum, g1var, gmax_s, den_s, num_s):
    ph = pl.program_id(0)
    b = pl.program_id(1)
    neg = jnp.float32(-1e30)

    def y_block():
        invdeg = 1.0 / (deg_ref[:, 0:1] + 1.0)
        return p_ref[...] + invdeg * z_ref[...] + b3[...]

    def mask():
        iota_g = lax.broadcasted_iota(jnp.int32, (N_GRAPHS, _TB), 0)
        return batch_ref[...].reshape(1, _TB) == iota_g

    def gate_pre(h):
        return h @ w_g1[...] + b_g1[...]

    @pl.when((ph == 0) & (b == 0))
    def _():
        ysum[...] = jnp.zeros_like(ysum)
        yvar[...] = jnp.zeros_like(yvar)
        g1sum[...] = jnp.zeros_like(g1sum)
        g1var[...] = jnp.zeros_like(g1var)
        gmax_s[...] = jnp.full_like(gmax_s, neg)
        den_s[...] = jnp.zeros_like(den_s)
        num_s[...] = jnp.zeros_like(num_s)

    @pl.when(ph == 0)
    def _():
        ysum[...] += jnp.sum(y_block(), axis=0, keepdims=True)

    @pl.when(ph == 1)
    def _():
        m = ysum[...] / _FN
        yvar[...] += jnp.sum(jnp.square(y_block() - m), axis=0, keepdims=True)

    @pl.when(ph == 2)
    def _():
        m = ysum[...] / _FN
        v = yvar[...] / _FN
        h = _elu(_bn_apply(y_block(), m, v, g3[...], be3[...]))
        h_scr[pl.ds(b * _TB, _TB), :] = h
        g1sum[...] += jnp.sum(gate_pre(h), axis=0, keepdims=True)

    @pl.when(ph == 3)
    def _():
        g1 = gate_pre(h_scr[pl.ds(b * _TB, _TB), :])
        m1 = g1sum[...] / _FN
        g1var[...] += jnp.sum(jnp.square(g1 - m1), axis=0, keepdims=True)

    @pl.when(ph == 4)
    def _():
        g1 = gate_pre(h_scr[pl.ds(b * _TB, _TB), :])
        m1 = g1sum[...] / _FN
        v1 = g1var[...] / _FN
        g1 = jnp.maximum(_bn_apply(g1, m1, v1, g_gbn[...], be_gbn[...]), 0.0)
        g = g1 @ w_g2[...] + b_g2[...]
        g_scr[pl.ds(b * _TB, _TB), :] = g
        gt = g.reshape(1, _TB)
        cand = jnp.max(jnp.where(mask(), gt, neg), axis=1, keepdims=True)
        gmax_s[...] = jnp.maximum(gmax_s[...], cand)

    @pl.when(ph == 5)
    def _():
        gt = g_scr[pl.ds(b * _TB, _TB), :].reshape(1, _TB)
        e = jnp.exp(jnp.where(mask(), gt - gmax_s[...], neg))
        den_s[...] += jnp.sum(e, axis=1, keepdims=True)
        num_s[...] += jnp.dot(e, h_scr[pl.ds(b * _TB, _TB), :],
                              preferred_element_type=jnp.float32)

    @pl.when((ph == 6) & (b == 0))
    def _():
        pooled = num_s[...] / (den_s[...] + 1e-16)
        o = pooled @ w_d1[...] + b_d1[...]
        mo, vo = _bn_stats(o)
        o = _elu(_bn_apply(o, mo, vo, g_dbn[...], be_dbn[...]))
        out_ref[...] = o @ w_d2[...] + b_d2[...]


def _t3_call(p3, z3, deg, batch2d, prm):
    blk = lambda w: pl.BlockSpec((_TB, w), lambda ph, b: (b, 0))
    full = lambda s: pl.BlockSpec(s, lambda ph, b: (0, 0))
    param_shapes = [a.shape for a in prm]
    return pl.pallas_call(
        _t3_body,
        grid=(7, _NB),
        in_specs=[blk(128), blk(128), blk(16),
                  pl.BlockSpec((1, 1, _TB), lambda ph, b: (b, 0, 0))]
                 + [full(s) for s in param_shapes],
        out_specs=full((N_GRAPHS, 32)),
        out_shape=jax.ShapeDtypeStruct((N_GRAPHS, 32), jnp.float32),
        scratch_shapes=[
            pltpu.VMEM((N_NODES, 128), jnp.float32),
            pltpu.VMEM((N_NODES, 1), jnp.float32),
            pltpu.VMEM((1, 128), jnp.float32),
            pltpu.VMEM((1, 128), jnp.float32),
            pltpu.VMEM((1, 42), jnp.float32),
            pltpu.VMEM((1, 42), jnp.float32),
            pltpu.VMEM((N_GRAPHS, 1), jnp.float32),
            pltpu.VMEM((N_GRAPHS, 1), jnp.float32),
            pltpu.VMEM((N_GRAPHS, 128), jnp.float32),
        ],
    )(p3, z3, deg, batch2d, *prm)


def _row(a):
    return a.reshape(1, -1)


def _pad_edges(a, mode='2d'):
    a = jnp.concatenate([a, jnp.zeros((E_PAD - a.shape[0],), a.dtype)])
    if mode == 'flat':
        return a.reshape(NS, EPT)
    return a.reshape(NS, NCHUNK, CHUNK)


def kernel(x, edge_index, edge_weight, batch, params):
    srcs = _pad_edges(edge_index[0])
    dsts = _pad_edges(edge_index[1])
    ews = _pad_edges(edge_weight, mode='flat')

    pa, pc = params['a'], params['c']

    z1a, z1c = pl.pallas_call(
        _t0_body,
        out_shape=[jax.ShapeDtypeStruct((N_PAD, 32), jnp.float32)] * 2,
    )(x, _row(pa['bn0'][0]), _row(pa['bn0'][1]), pa['gcn1_W'],
      _row(pc['bn0'][0]), _row(pc['bn0'][1]), pc['gcn1_W'])

    zeros32 = jnp.zeros((N_PAD, 32), jnp.float32)
    zeros16 = jnp.zeros((N_PAD, 16), jnp.float32)
    p1a, p1c, deg, norms = _SPMM1(z1a, z1c, srcs, dsts, ews,
                                  zeros32, zeros16)

    def tmid(p1, p2, w, pA, pB, bnname, wname):
        return pl.pallas_call(
            _tmid_body,
            out_shape=[jax.ShapeDtypeStruct((N_PAD, w), jnp.float32)] * 2,
        )(p1[0], p1[1], p2[0], p2[1], deg,
          _row(pA[bnname.replace('bn', 'gcn') + '_b']),
          _row(pA[bnname][0]), _row(pA[bnname][1]), pA[wname],
          _row(pB[bnname.replace('bn', 'gcn') + '_b']),
          _row(pB[bnname][0]), _row(pB[bnname][1]), pB[wname])

    z2a, z2c = tmid((p1a, p1c), (z1a, z1c), 64, pa, pc, 'bn1', 'gcn2_W')
    zeros64 = jnp.zeros((N_PAD, 64), jnp.float32)
    p2a, p2c = _SPMM[64](z2a, z2c, srcs, dsts, norms, zeros64)

    z3a, z3c = tmid((p2a, p2c), (z2a, z2c), 128, pa, pc, 'bn2', 'gcn3_W')
    zeros128 = jnp.zeros((N_PAD, 128), jnp.float32)
    p3a, p3c = _SPMM[128](z3a, z3c, srcs, dsts, norms, zeros128)

    def t3_params(p):
        return (
            _row(p['gcn3_b']), _row(p['bn3'][0]), _row(p['bn3'][1]),
            p['gate1_W'], _row(p['gate1_b']),
            _row(p['gate_bn'][0]), _row(p['gate_bn'][1]),
            p['gate2_W'], _row(p['gate2_b']),
            p['dnn1_W'], _row(p['dnn1_b']),
            _row(p['dnn_bn'][0]), _row(p['dnn_bn'][1]),
            p['dnn2_W'], _row(p['dnn2_b']),
        )

    batch2d = batch.reshape(_NB, 1, _TB)
    a_out = _t3_call(p3a, z3a, deg, batch2d, t3_params(pa))
    c_out = _t3_call(p3c, z3c, deg, batch2d, t3_params(pc))

    return (a_out, c_out)
